# baseline (speedup 1.0000x reference)
import functools

import jax
import jax.numpy as jnp
from jax import lax
from jax.experimental import pallas as pl
from jax.experimental.pallas import tpu as pltpu
from jax.experimental.pallas import tpu_sc as plsc

D = 128
LANES = 16


def _make_sc_kernel(N, NC, NS, C):
    NW = NC * NS
    per_w = N // NW
    steps = per_w // C
    G = C // LANES
    assert steps % 2 == 0 and C % LANES == 0
    mesh = plsc.VectorSubcoreMesh(core_axis_name="c", subcore_axis_name="s")

    row_shapes = [((C, D), jnp.float32), ((C, D), jnp.float32),
                  ((C, D), jnp.float32), ((C, D), jnp.float32),
                  ((C, 2 * D), jnp.int32), ((C, D), jnp.int32)]

    @functools.partial(
        pl.kernel,
        out_type=jax.ShapeDtypeStruct((N,), jnp.float32),
        mesh=mesh,
        compiler_params=pltpu.CompilerParams(needs_layout_passes=False),
        scratch_types=(
            [pltpu.VMEM((4 * C,), jnp.int32) for _ in range(2)]
            + [pltpu.VMEM(sh, dt) for sh, dt in row_shapes] * 2
            + [pltpu.VMEM((C,), jnp.float32) for _ in range(2)]
            + [pltpu.VMEM((LANES, LANES), jnp.float32)]
            + [pltpu.SemaphoreType.DMA] * 6
        ),
    )
    def sc_kernel(idx_hbm, e_re_hbm, e_im_hbm, r4_hbm, t2_hbm, out_hbm,
                  idx0, idx1,
                  sre0, sim0, ore0, oim0, r40, t20,
                  sre1, sim1, ore1, oim1, r41, t21,
                  outv0, outv1, m_v,
                  sem_g0, sem_g1, sem_i0, sem_i1, sem_o0, sem_o1):
        wid = lax.axis_index("s") * NC + lax.axis_index("c")
        base = wid * per_w

        sets = [
            dict(idx=idx0, rows=[sre0, sim0, ore0, oim0, r40, t20],
                 outv=outv0, sem_g=sem_g0, sem_i=sem_i0, sem_o=sem_o0),
            dict(idx=idx1, rows=[sre1, sim1, ore1, oim1, r41, t21],
                 outv=outv1, sem_g=sem_g1, sem_i=sem_i1, sem_o=sem_o1),
        ]

        def fire_gathers(st):
            idx = st["idx"]
            rows = st["rows"]
            s_i = idx.at[pl.ds(0, C)]
            r_i = idx.at[pl.ds(C, C)]
            o_i = idx.at[pl.ds(2 * C, C)]
            t_i = idx.at[pl.ds(3 * C, C)]
            pltpu.async_copy(e_re_hbm.at[s_i], rows[0], st["sem_g"])
            pltpu.async_copy(e_im_hbm.at[s_i], rows[1], st["sem_g"])
            pltpu.async_copy(e_re_hbm.at[o_i], rows[2], st["sem_g"])
            pltpu.async_copy(e_im_hbm.at[o_i], rows[3], st["sem_g"])
            pltpu.async_copy(r4_hbm.at[r_i], rows[4], st["sem_g"])
            pltpu.async_copy(t2_hbm.at[t_i], rows[5], st["sem_g"])

        def drain_gathers(st):
            srcs = [e_re_hbm, e_im_hbm, e_re_hbm, e_im_hbm, r4_hbm, t2_hbm]
            for src, dst in zip(srcs, st["rows"]):
                pltpu.make_async_copy(src.at[pl.ds(0, C)], dst,
                                      st["sem_g"]).wait()

        def compute(st):
            sre_v, sim_v, ore_v, oim_v, r4_v, t2_v = st["rows"]
            outv = st["outv"]
            lane = lax.iota(jnp.int32, LANES)

            def group(grp, carry):
                row0 = pl.multiple_of(grp * LANES, LANES)

                def elem(e, carry2):
                    row = row0 + e

                    def ldp(ref, off):
                        words = ref[row, pl.ds(off, LANES)]
                        return plsc.unpack(
                            plsc.bitcast(words, jnp.bfloat16),
                            format=plsc.PackFormat.INTERLEAVED)

                    acc = jnp.zeros((LANES,), jnp.float32)
                    for k in range(D // LANES):
                        c0 = LANES * k
                        sre = sre_v[row, pl.ds(c0, LANES)]
                        sim = sim_v[row, pl.ds(c0, LANES)]
                        ore = ore_v[row, pl.ds(c0, LANES)]
                        oim = oim_v[row, pl.ds(c0, LANES)]
                        rre, rim = ldp(r4_v, c0)
                        rnre, rnim = ldp(r4_v, c0 + D)
                        tre, tim = ldp(t2_v, c0)
                        rrt = rre * tre - rim * tim + rnre
                        rit = rre * tim + rim * tre + rnim
                        a = sre * ore + sim * oim
                        b = sre * oim - sim * ore
                        acc = acc + a * rrt + b * rit
                    m_v[e, :] = acc
                    return carry2

                lax.fori_loop(0, LANES, elem, 0, unroll=4)
                tot = plsc.load_gather(
                    m_v, [lane, jnp.zeros((LANES,), jnp.int32)])
                for l in range(1, LANES):
                    tot = tot + plsc.load_gather(
                        m_v, [lane, jnp.full((LANES,), l, jnp.int32)])
                outv[pl.ds(row0, LANES)] = tot
                return carry

            lax.fori_loop(0, G, group, 0)

        def step(g, p):
            st = sets[p]
            st_n = sets[1 - p]
            off = base + g * C
            drain_gathers(st)
            q = wid * steps + g
            @pl.when(g + 2 < steps)
            def _prefetch_idx():
                pltpu.async_copy(
                    idx_hbm.at[pl.ds((q + 2) * 4 * C, 4 * C)], st["idx"],
                    st["sem_i"])
            @pl.when(g + 1 < steps)
            def _fire_next():
                pltpu.make_async_copy(
                    idx_hbm.at[pl.ds(0, 4 * C)], st_n["idx"],
                    st_n["sem_i"]).wait()
                fire_gathers(st_n)
            @pl.when(g >= 2)
            def _drain_out():
                pltpu.make_async_copy(
                    st["outv"], out_hbm.at[pl.ds(off, C)], st["sem_o"]).wait()
            compute(st)
            pltpu.async_copy(st["outv"], out_hbm.at[pl.ds(off, C)],
                             st["sem_o"])

        q0 = wid * steps
        pltpu.sync_copy(idx_hbm.at[pl.ds(q0 * 4 * C, 4 * C)], sets[0]["idx"])
        fire_gathers(sets[0])
        pltpu.async_copy(idx_hbm.at[pl.ds((q0 + 1) * 4 * C, 4 * C)],
                         sets[1]["idx"], sets[1]["sem_i"])

        def pair(i, carry):
            step(2 * i, 0)
            step(2 * i + 1, 1)
            return carry

        lax.fori_loop(0, steps // 2, pair, 0)

        pltpu.make_async_copy(sets[0]["outv"],
                              out_hbm.at[pl.ds(base, C)], sem_o0).wait()
        pltpu.make_async_copy(sets[1]["outv"],
                              out_hbm.at[pl.ds(base, C)], sem_o1).wait()

    return sc_kernel


def kernel(s, r, o, t, E_re, E_im, R_re, R_im, R_no_time_re, R_no_time_im,
           T_re, T_im):
    B, L = s.shape
    N = B * L
    si = s.reshape(N).astype(jnp.int32)
    ri = r.reshape(N).astype(jnp.int32)
    oi = o.reshape(N).astype(jnp.int32)
    ti = t[:, :, 0].reshape(N).astype(jnp.int32)
    C = 32
    idx4 = (jnp.stack([si, ri, oi, ti])
            .reshape(4, N // C, C).transpose(1, 0, 2).reshape(-1))
    def pack(re, im):
        lo = jax.lax.bitcast_convert_type(
            re.astype(jnp.bfloat16), jnp.uint16).astype(jnp.uint32)
        hi = jax.lax.bitcast_convert_type(
            im.astype(jnp.bfloat16), jnp.uint16).astype(jnp.uint32)
        return jax.lax.bitcast_convert_type(lo | (hi << 16), jnp.int32)

    r4 = pack(jnp.concatenate([R_re, R_no_time_re], axis=1),
              jnp.concatenate([R_im, R_no_time_im], axis=1))
    t2 = pack(T_re, T_im)
    info = plsc.get_sparse_core_info()
    fn = _make_sc_kernel(N, info.num_cores, info.num_subcores, C)
    out = fn(idx4, E_re, E_im, r4, t2)
    return out.reshape(B, L)


# speedup vs baseline: 1.0982x; 1.0982x over previous
import functools

import jax
import jax.numpy as jnp
from jax import lax
from jax.experimental import pallas as pl
from jax.experimental.pallas import tpu as pltpu
from jax.experimental.pallas import tpu_sc as plsc

D = 128
LANES = 16


def _make_sc_kernel(N, NC, NS, C):
    NW = NC * NS
    per_w = N // NW
    steps = per_w // C
    G = C // LANES
    assert steps % 2 == 0 and C % LANES == 0
    mesh = plsc.VectorSubcoreMesh(core_axis_name="c", subcore_axis_name="s")

    row_shapes = [((C, D), jnp.float32), ((C, D), jnp.float32),
                  ((C, D), jnp.float32), ((C, D), jnp.float32),
                  ((C, 2 * D), jnp.int32), ((C, D), jnp.int32)]

    @functools.partial(
        pl.kernel,
        out_type=jax.ShapeDtypeStruct((N,), jnp.float32),
        mesh=mesh,
        compiler_params=pltpu.CompilerParams(needs_layout_passes=False),
        scratch_types=(
            [pltpu.VMEM((4 * C,), jnp.int32) for _ in range(2)]
            + [pltpu.VMEM(sh, dt) for sh, dt in row_shapes] * 2
            + [pltpu.VMEM((C,), jnp.float32) for _ in range(2)]
            + [pltpu.VMEM((LANES, LANES), jnp.float32)]
            + [pltpu.SemaphoreType.DMA] * 6
        ),
    )
    def sc_kernel(idx_hbm, e_re_hbm, e_im_hbm, r4_hbm, t2_hbm, out_hbm,
                  idx0, idx1,
                  sre0, sim0, ore0, oim0, r40, t20,
                  sre1, sim1, ore1, oim1, r41, t21,
                  outv0, outv1, m_v,
                  sem_g0, sem_g1, sem_i0, sem_i1, sem_o0, sem_o1):
        wid = lax.axis_index("s") * NC + lax.axis_index("c")
        base = wid * per_w

        sets = [
            dict(idx=idx0, rows=[sre0, sim0, ore0, oim0, r40, t20],
                 outv=outv0, sem_g=sem_g0, sem_i=sem_i0, sem_o=sem_o0),
            dict(idx=idx1, rows=[sre1, sim1, ore1, oim1, r41, t21],
                 outv=outv1, sem_g=sem_g1, sem_i=sem_i1, sem_o=sem_o1),
        ]

        def fire_gathers(st):
            idx = st["idx"]
            rows = st["rows"]
            s_i = idx.at[pl.ds(0, C)]
            r_i = idx.at[pl.ds(C, C)]
            o_i = idx.at[pl.ds(2 * C, C)]
            t_i = idx.at[pl.ds(3 * C, C)]
            pltpu.async_copy(e_re_hbm.at[s_i], rows[0], st["sem_g"])
            pltpu.async_copy(e_im_hbm.at[s_i], rows[1], st["sem_g"])
            pltpu.async_copy(e_re_hbm.at[o_i], rows[2], st["sem_g"])
            pltpu.async_copy(e_im_hbm.at[o_i], rows[3], st["sem_g"])
            pltpu.async_copy(r4_hbm.at[r_i], rows[4], st["sem_g"])
            pltpu.async_copy(t2_hbm.at[t_i], rows[5], st["sem_g"])

        def drain_gathers(st):
            srcs = [e_re_hbm, e_im_hbm, e_re_hbm, e_im_hbm, r4_hbm, t2_hbm]
            for src, dst in zip(srcs, st["rows"]):
                pltpu.make_async_copy(src.at[pl.ds(0, C)], dst,
                                      st["sem_g"]).wait()

        def compute(st):
            sre_v, sim_v, ore_v, oim_v, r4_v, t2_v = st["rows"]
            outv = st["outv"]
            lane = lax.iota(jnp.int32, LANES)

            def group(grp, carry):
                row0 = pl.multiple_of(grp * LANES, LANES)

                def elem(e, carry2):
                    row = row0 + e

                    def ldp(ref, off):
                        words = ref[row, pl.ds(off, LANES)]
                        return plsc.unpack(
                            plsc.bitcast(words, jnp.bfloat16),
                            format=plsc.PackFormat.INTERLEAVED)

                    acc = jnp.zeros((LANES,), jnp.float32)
                    for k in range(D // LANES):
                        c0 = LANES * k
                        sre = sre_v[row, pl.ds(c0, LANES)]
                        sim = sim_v[row, pl.ds(c0, LANES)]
                        ore = ore_v[row, pl.ds(c0, LANES)]
                        oim = oim_v[row, pl.ds(c0, LANES)]
                        rre, rim = ldp(r4_v, c0)
                        rnre, rnim = ldp(r4_v, c0 + D)
                        tre, tim = ldp(t2_v, c0)
                        rrt = rre * tre - rim * tim + rnre
                        rit = rre * tim + rim * tre + rnim
                        a = sre * ore + sim * oim
                        b = sre * oim - sim * ore
                        acc = acc + a * rrt + b * rit
                    m_v[e, :] = acc
                    return carry2

                lax.fori_loop(0, LANES, elem, 0, unroll=4)
                tot = plsc.load_gather(
                    m_v, [lane, jnp.zeros((LANES,), jnp.int32)])
                for l in range(1, LANES):
                    tot = tot + plsc.load_gather(
                        m_v, [lane, jnp.full((LANES,), l, jnp.int32)])
                outv[pl.ds(row0, LANES)] = tot
                return carry

            lax.fori_loop(0, G, group, 0)

        def step(g, p):
            st = sets[p]
            st_n = sets[1 - p]
            off = base + g * C
            drain_gathers(st)
            q = wid * steps + g
            @pl.when(g + 2 < steps)
            def _prefetch_idx():
                pltpu.async_copy(
                    idx_hbm.at[pl.ds((q + 2) * 4 * C, 4 * C)], st["idx"],
                    st["sem_i"])
            @pl.when(g + 1 < steps)
            def _fire_next():
                pltpu.make_async_copy(
                    idx_hbm.at[pl.ds(0, 4 * C)], st_n["idx"],
                    st_n["sem_i"]).wait()
                fire_gathers(st_n)
            @pl.when(g >= 2)
            def _drain_out():
                pltpu.make_async_copy(
                    st["outv"], out_hbm.at[pl.ds(off, C)], st["sem_o"]).wait()
            compute(st)
            pltpu.async_copy(st["outv"], out_hbm.at[pl.ds(off, C)],
                             st["sem_o"])

        q0 = wid * steps
        pltpu.sync_copy(idx_hbm.at[pl.ds(q0 * 4 * C, 4 * C)], sets[0]["idx"])
        fire_gathers(sets[0])
        pltpu.async_copy(idx_hbm.at[pl.ds((q0 + 1) * 4 * C, 4 * C)],
                         sets[1]["idx"], sets[1]["sem_i"])

        def pair(i, carry):
            step(2 * i, 0)
            step(2 * i + 1, 1)
            return carry

        lax.fori_loop(0, steps // 2, pair, 0)

        pltpu.make_async_copy(sets[0]["outv"],
                              out_hbm.at[pl.ds(base, C)], sem_o0).wait()
        pltpu.make_async_copy(sets[1]["outv"],
                              out_hbm.at[pl.ds(base, C)], sem_o1).wait()

    return sc_kernel


def kernel(s, r, o, t, E_re, E_im, R_re, R_im, R_no_time_re, R_no_time_im,
           T_re, T_im):
    B, L = s.shape
    N = B * L
    si = s.reshape(N).astype(jnp.int32)
    ri = r.reshape(N).astype(jnp.int32)
    oi = o.reshape(N).astype(jnp.int32)
    ti = t[:, :, 0].reshape(N).astype(jnp.int32)
    C = 64
    idx4 = (jnp.stack([si, ri, oi, ti])
            .reshape(4, N // C, C).transpose(1, 0, 2).reshape(-1))
    def pack(re, im):
        lo = jax.lax.bitcast_convert_type(
            re.astype(jnp.bfloat16), jnp.uint16).astype(jnp.uint32)
        hi = jax.lax.bitcast_convert_type(
            im.astype(jnp.bfloat16), jnp.uint16).astype(jnp.uint32)
        return jax.lax.bitcast_convert_type(lo | (hi << 16), jnp.int32)

    r4 = pack(jnp.concatenate([R_re, R_no_time_re], axis=1),
              jnp.concatenate([R_im, R_no_time_im], axis=1))
    t2 = pack(T_re, T_im)
    info = plsc.get_sparse_core_info()
    fn = _make_sc_kernel(N, info.num_cores, info.num_subcores, C)
    out = fn(idx4, E_re, E_im, r4, t2)
    return out.reshape(B, L)
